# baseline (speedup 1.0000x reference)
import functools

import jax
import jax.numpy as jnp
from jax.experimental import pallas as pl
from jax.experimental.pallas import tpu as pltpu

_TEMP_INV = 10.0
_B = 1024
_D = 32
_M = 100000
_TM = 2048


def _fused_kernel(labels_ref, f_ref, c_ref, out_ref, ups_ref, down_ref):
    i = pl.program_id(0)
    nt = pl.num_programs(0)

    @pl.when(i == 0)
    def _init():
        ups_ref[...] = jnp.zeros_like(ups_ref)
        down_ref[...] = jnp.zeros_like(down_ref)

    f = f_ref[...]
    nrm = jnp.sqrt(jnp.sum(f * f, axis=1, keepdims=True))
    f = f / jnp.maximum(nrm, 1e-12)
    c = c_ref[...]
    g = jax.lax.dot_general(
        c, f, (((1,), (1,)), ((), ())), preferred_element_type=jnp.float32
    )
    e = jnp.exp(g * _TEMP_INV)
    row_id = i * _TM + jax.lax.broadcasted_iota(jnp.int32, (_TM, _B), 0)
    lbl = labels_ref[...]
    ups_ref[...] += jnp.sum(jnp.where(row_id == lbl, e, 0.0), axis=0, keepdims=True)
    down_ref[...] += jnp.sum(e, axis=0, keepdims=True)

    @pl.when(i == nt - 1)
    def _fin():
        n_pad = pl.cdiv(_M, _TM) * _TM - _M
        out_ref[...] = ups_ref[...] / (down_ref[...] - float(n_pad))


@functools.partial(jax.jit, static_argnames=())
def kernel(features, labels, centrals):
    m_pad = pl.cdiv(_M, _TM) * _TM
    c_pad = jnp.pad(centrals, ((0, m_pad - _M), (0, 0)))
    labels2d = labels.reshape(1, _B)
    nt = m_pad // _TM
    out = pl.pallas_call(
        _fused_kernel,
        grid=(nt,),
        in_specs=[
            pl.BlockSpec((1, _B), lambda i: (0, 0)),
            pl.BlockSpec((_B, _D), lambda i: (0, 0)),
            pl.BlockSpec((_TM, _D), lambda i: (i, 0)),
        ],
        out_specs=pl.BlockSpec((1, _B), lambda i: (0, 0)),
        out_shape=jax.ShapeDtypeStruct((1, _B), jnp.float32),
        scratch_shapes=[
            pltpu.VMEM((1, _B), jnp.float32),
            pltpu.VMEM((1, _B), jnp.float32),
        ],
    )(labels2d, features, c_pad)
    return out.reshape(_B)


# speedup vs baseline: 1.2126x; 1.2126x over previous
import functools

import jax
import jax.numpy as jnp
from jax.experimental import pallas as pl
from jax.experimental.pallas import tpu as pltpu

_TEMP_INV = 10.0
_LOG2E = 1.4426950408889634
_B = 1024
_D = 32
_M = 100000
_TM = 2048


def _fused_kernel(f_ref, c_ref, lrows_ref, out_ref, f_scr, down_ref):
    i = pl.program_id(0)
    nt = pl.num_programs(0)

    @pl.when(i == 0)
    def _init():
        f = f_ref[...]
        nrm = jnp.sqrt(jnp.sum(f * f, axis=1, keepdims=True))
        f_scr[...] = f / jnp.maximum(nrm, 1e-12)
        down_ref[...] = jnp.zeros_like(down_ref)

    f = f_scr[...]
    c = c_ref[...]
    g = jax.lax.dot_general(
        c, f, (((1,), (1,)), ((), ())), preferred_element_type=jnp.float32
    )
    e = jnp.exp2(g * (_TEMP_INV * _LOG2E))
    down_ref[...] += jnp.sum(e, axis=0, keepdims=True)

    @pl.when(i == nt - 1)
    def _fin():
        fb = f.astype(jnp.bfloat16).astype(jnp.float32)
        rb = lrows_ref[...].astype(jnp.bfloat16).astype(jnp.float32)
        gl = jnp.sum(fb * rb, axis=1, keepdims=True)
        ups = jnp.exp2(gl.reshape(1, _B) * (_TEMP_INV * _LOG2E))
        n_pad = pl.cdiv(_M, _TM) * _TM - _M
        out_ref[...] = ups / (down_ref[...] - float(n_pad))


@functools.partial(jax.jit, static_argnames=())
def kernel(features, labels, centrals):
    m_pad = pl.cdiv(_M, _TM) * _TM
    c_pad = jnp.pad(centrals, ((0, m_pad - _M), (0, 0)))
    lrows = jnp.take(centrals, labels, axis=0)
    nt = m_pad // _TM
    out = pl.pallas_call(
        _fused_kernel,
        grid=(nt,),
        in_specs=[
            pl.BlockSpec((_B, _D), lambda i: (0, 0)),
            pl.BlockSpec((_TM, _D), lambda i: (i, 0)),
            pl.BlockSpec((_B, _D), lambda i: (0, 0)),
        ],
        out_specs=pl.BlockSpec((1, _B), lambda i: (0, 0)),
        out_shape=jax.ShapeDtypeStruct((1, _B), jnp.float32),
        scratch_shapes=[
            pltpu.VMEM((_B, _D), jnp.float32),
            pltpu.VMEM((1, _B), jnp.float32),
        ],
    )(features, c_pad, lrows)
    return out.reshape(_B)


# speedup vs baseline: 1.5952x; 1.3156x over previous
import functools

import jax
import jax.numpy as jnp
from jax.experimental import pallas as pl
from jax.experimental.pallas import tpu as pltpu

_TEMP_INV = 10.0
_LOG2E = 1.4426950408889634
_B = 1024
_D = 32
_M = 100000
_TM = 2000


def _fused_kernel(ft_ref, c_ref, lrowst_ref, out_ref, ft_scr, down_ref):
    i = pl.program_id(0)
    nt = pl.num_programs(0)

    @pl.when(i == 0)
    def _init():
        ft = ft_ref[...]
        nrm = jnp.sqrt(jnp.sum(ft * ft, axis=0, keepdims=True))
        ft_scr[...] = ft / jnp.maximum(nrm, 1e-12)
        down_ref[...] = jnp.zeros_like(down_ref)

    ft = ft_scr[...]
    c = c_ref[...]
    g = jax.lax.dot_general(
        c, ft, (((1,), (0,)), ((), ())), preferred_element_type=jnp.float32
    )
    e = jnp.exp2(g * (_TEMP_INV * _LOG2E))
    down_ref[...] += jnp.sum(e, axis=0, keepdims=True)

    @pl.when(i == nt - 1)
    def _fin():
        fb = ft.astype(jnp.bfloat16).astype(jnp.float32)
        rb = lrowst_ref[...].astype(jnp.bfloat16).astype(jnp.float32)
        gl = jnp.sum(fb * rb, axis=0, keepdims=True)
        ups = jnp.exp2(gl * (_TEMP_INV * _LOG2E))
        out_ref[...] = ups / down_ref[...]


@functools.partial(jax.jit, static_argnames=())
def kernel(features, labels, centrals):
    ft = features.T
    lrowst = jnp.take(centrals, labels, axis=0).T
    nt = _M // _TM
    out = pl.pallas_call(
        _fused_kernel,
        grid=(nt,),
        in_specs=[
            pl.BlockSpec((_D, _B), lambda i: (0, 0)),
            pl.BlockSpec((_TM, _D), lambda i: (i, 0)),
            pl.BlockSpec((_D, _B), lambda i: (0, 0)),
        ],
        out_specs=pl.BlockSpec((1, _B), lambda i: (0, 0)),
        out_shape=jax.ShapeDtypeStruct((1, _B), jnp.float32),
        scratch_shapes=[
            pltpu.VMEM((_D, _B), jnp.float32),
            pltpu.VMEM((1, _B), jnp.float32),
        ],
    )(ft, centrals, lrowst)
    return out.reshape(_B)


# speedup vs baseline: 1.6741x; 1.0494x over previous
import functools

import jax
import jax.numpy as jnp
from jax.experimental import pallas as pl
from jax.experimental.pallas import tpu as pltpu

_TEMP_INV = 10.0
_LOG2E = 1.4426950408889634
_B = 1024
_D = 32
_M = 100000
_TM = 4000


def _fused_kernel(ft_ref, c_ref, lrowst_ref, out_ref, ft_scr, down_ref):
    i = pl.program_id(0)
    nt = pl.num_programs(0)

    @pl.when(i == 0)
    def _init():
        ft = ft_ref[...]
        nrm = jnp.sqrt(jnp.sum(ft * ft, axis=0, keepdims=True))
        ft_scr[...] = ft / jnp.maximum(nrm, 1e-12)
        down_ref[...] = jnp.zeros_like(down_ref)

    ft = ft_scr[...]
    c = c_ref[...]
    g = jax.lax.dot_general(
        c, ft, (((1,), (0,)), ((), ())), preferred_element_type=jnp.float32
    )
    e = jnp.exp2(g * (_TEMP_INV * _LOG2E))
    down_ref[...] += jnp.sum(e, axis=0, keepdims=True)

    @pl.when(i == nt - 1)
    def _fin():
        fb = ft.astype(jnp.bfloat16).astype(jnp.float32)
        rb = lrowst_ref[...].astype(jnp.bfloat16).astype(jnp.float32)
        gl = jnp.sum(fb * rb, axis=0, keepdims=True)
        ups = jnp.exp2(gl * (_TEMP_INV * _LOG2E))
        out_ref[...] = ups / down_ref[...]


@functools.partial(jax.jit, static_argnames=())
def kernel(features, labels, centrals):
    ft = features.T
    lrowst = jnp.take(centrals, labels, axis=0).T
    nt = _M // _TM
    out = pl.pallas_call(
        _fused_kernel,
        grid=(nt,),
        in_specs=[
            pl.BlockSpec((_D, _B), lambda i: (0, 0)),
            pl.BlockSpec((_TM, _D), lambda i: (i, 0)),
            pl.BlockSpec((_D, _B), lambda i: (0, 0)),
        ],
        out_specs=pl.BlockSpec((1, _B), lambda i: (0, 0)),
        out_shape=jax.ShapeDtypeStruct((1, _B), jnp.float32),
        scratch_shapes=[
            pltpu.VMEM((_D, _B), jnp.float32),
            pltpu.VMEM((1, _B), jnp.float32),
        ],
    )(ft, centrals, lrowst)
    return out.reshape(_B)


# speedup vs baseline: 1.7096x; 1.0212x over previous
import functools

import jax
import jax.numpy as jnp
from jax.experimental import pallas as pl
from jax.experimental.pallas import tpu as pltpu

_TEMP_INV = 10.0
_LOG2E = 1.4426950408889634
_B = 1024
_D = 32
_M = 100000
_TM = 10000


def _fused_kernel(ft_ref, c_ref, lrowst_ref, out_ref, ft_scr, down_ref):
    i = pl.program_id(0)
    nt = pl.num_programs(0)

    @pl.when(i == 0)
    def _init():
        ft = ft_ref[...]
        nrm = jnp.sqrt(jnp.sum(ft * ft, axis=0, keepdims=True))
        ft_scr[...] = ft / jnp.maximum(nrm, 1e-12)
        down_ref[...] = jnp.zeros_like(down_ref)

    ft = ft_scr[...]
    c = c_ref[...]
    g = jax.lax.dot_general(
        c, ft, (((1,), (0,)), ((), ())), preferred_element_type=jnp.float32
    )
    e = jnp.exp2(g * (_TEMP_INV * _LOG2E))
    down_ref[...] += jnp.sum(e, axis=0, keepdims=True)

    @pl.when(i == nt - 1)
    def _fin():
        fb = ft.astype(jnp.bfloat16).astype(jnp.float32)
        rb = lrowst_ref[...].astype(jnp.bfloat16).astype(jnp.float32)
        gl = jnp.sum(fb * rb, axis=0, keepdims=True)
        ups = jnp.exp2(gl * (_TEMP_INV * _LOG2E))
        out_ref[...] = ups / down_ref[...]


@functools.partial(jax.jit, static_argnames=())
def kernel(features, labels, centrals):
    ft = features.T
    lrowst = jnp.take(centrals, labels, axis=0).T
    nt = _M // _TM
    out = pl.pallas_call(
        _fused_kernel,
        grid=(nt,),
        in_specs=[
            pl.BlockSpec((_D, _B), lambda i: (0, 0)),
            pl.BlockSpec((_TM, _D), lambda i: (i, 0)),
            pl.BlockSpec((_D, _B), lambda i: (0, 0)),
        ],
        out_specs=pl.BlockSpec((1, _B), lambda i: (0, 0)),
        out_shape=jax.ShapeDtypeStruct((1, _B), jnp.float32),
        scratch_shapes=[
            pltpu.VMEM((_D, _B), jnp.float32),
            pltpu.VMEM((1, _B), jnp.float32),
        ],
    )(ft, centrals, lrowst)
    return out.reshape(_B)
